# baseline (speedup 1.0000x reference)
import functools
import jax
import jax.numpy as jnp
from jax import lax
from jax.experimental import pallas as pl
from jax.experimental.pallas import tpu as pltpu
from jax.experimental.pallas import tpu_sc as plsc

H = 128
NC = 2
NS = 16
NW = NC * NS

F32 = jnp.float32


def _mesh():
    return plsc.VectorSubcoreMesh(
        core_axis_name="c", subcore_axis_name="s", num_cores=NC, num_subcores=NS
    )


def _mlp2_body(x_ref, w1_ref, b1_ref, w2_ref, b2_ref, o_ref):
    h = jnp.dot(x_ref[...], w1_ref[...], preferred_element_type=F32, precision=jax.lax.Precision.HIGHEST) + b1_ref[...]
    h = jnp.maximum(h, 0.0)
    o_ref[...] = jnp.dot(h, w2_ref[...], preferred_element_type=F32, precision=jax.lax.Precision.HIGHEST) + b2_ref[...]


def _mlp2(x, w1, b1, w2, b2, block):
    r, k = x.shape
    hh = w1.shape[1]
    f = w2.shape[1]
    return pl.pallas_call(
        _mlp2_body,
        grid=(r // block,),
        in_specs=[
            pl.BlockSpec((block, k), lambda i: (i, 0)),
            pl.BlockSpec((k, hh), lambda i: (0, 0)),
            pl.BlockSpec((1, hh), lambda i: (0, 0)),
            pl.BlockSpec((hh, f), lambda i: (0, 0)),
            pl.BlockSpec((1, f), lambda i: (0, 0)),
        ],
        out_specs=pl.BlockSpec((block, f), lambda i: (i, 0)),
        out_shape=jax.ShapeDtypeStruct((r, f), F32),
    )(x, w1, b1.reshape(1, -1), w2, b2.reshape(1, -1))


def _pre_body(x_ref, wa_ref, wb_ref, t1_ref, t2_ref):
    x = x_ref[...]
    t1_ref[...] = jnp.dot(x, wa_ref[...], preferred_element_type=F32, precision=jax.lax.Precision.HIGHEST).astype(jnp.bfloat16)
    t2_ref[...] = jnp.dot(x, wb_ref[...], preferred_element_type=F32, precision=jax.lax.Precision.HIGHEST)


def _pre(x, wa, wb, block=2000):
    n = x.shape[0]
    return pl.pallas_call(
        _pre_body,
        grid=(n // block,),
        in_specs=[
            pl.BlockSpec((block, H), lambda i: (i, 0)),
            pl.BlockSpec((H, 2 * H), lambda i: (0, 0)),
            pl.BlockSpec((H, H), lambda i: (0, 0)),
        ],
        out_specs=[
            pl.BlockSpec((block, 2 * H), lambda i: (i, 0)),
            pl.BlockSpec((block, H), lambda i: (i, 0)),
        ],
        out_shape=[
            jax.ShapeDtypeStruct((n, 2 * H), jnp.bfloat16),
            jax.ShapeDtypeStruct((n, H), F32),
        ],
    )(x, wa, wb)


def _edge_body(g1_ref, g2_ref, ea_ref, we_ref, c1_ref, w2_ref, b2_ref,
               ve_ref, d1_ref, v2_ref, v2b_ref, eanext_ref, m_ref):
    ea = ea_ref[...]
    g1 = g1_ref[...].astype(F32)
    g2 = g2_ref[...].astype(F32)
    h = g1[:, :H] + g2 + jnp.dot(ea, we_ref[...], preferred_element_type=F32, precision=jax.lax.Precision.HIGHEST) + c1_ref[...]
    h = jnp.maximum(h, 0.0)
    ea_new = jnp.dot(h, w2_ref[...], preferred_element_type=F32, precision=jax.lax.Precision.HIGHEST) + b2_ref[...]
    t = g1[:, H:] + jnp.dot(ea_new, ve_ref[...], preferred_element_type=F32, precision=jax.lax.Precision.HIGHEST) + d1_ref[...]
    t = jnp.maximum(t, 0.0)
    m_ref[...] = jnp.dot(t, v2_ref[...], preferred_element_type=F32, precision=jax.lax.Precision.HIGHEST) + v2b_ref[...]
    eanext_ref[...] = ea_new + ea


def _edge(g1, g2, ea, we, c1, w2, b2, ve, d1, v2, v2b, block=3200):
    e = ea.shape[0]
    wspec = pl.BlockSpec((H, H), lambda i: (0, 0))
    bspec = pl.BlockSpec((1, H), lambda i: (0, 0))
    return pl.pallas_call(
        _edge_body,
        grid=(e // block,),
        in_specs=[
            pl.BlockSpec((block, 2 * H), lambda i: (i, 0)),
            pl.BlockSpec((block, H), lambda i: (i, 0)),
            pl.BlockSpec((block, H), lambda i: (i, 0)),
            wspec, bspec, wspec, bspec, wspec, bspec, wspec, bspec,
        ],
        out_specs=[
            pl.BlockSpec((block, H), lambda i: (i, 0)),
            pl.BlockSpec((block, H), lambda i: (i, 0)),
        ],
        out_shape=[
            jax.ShapeDtypeStruct((e, H), F32),
            jax.ShapeDtypeStruct((e, H), F32),
        ],
    )(g1, g2, ea, we, c1.reshape(1, -1), w2, b2.reshape(1, -1),
      ve, d1.reshape(1, -1), v2, v2b.reshape(1, -1))


def _node_body(x_ref, s_ref, c_ref, cx_ref, ca_ref, e1_ref, u2_ref, u2b_ref, o_ref):
    s = s_ref[0] + s_ref[1]
    cnt = c_ref[0, :, 0:1] + c_ref[1, :, 0:1]
    agg = s / jnp.maximum(cnt, 1.0)
    x = x_ref[...]
    h = (jnp.dot(x, cx_ref[...], preferred_element_type=F32, precision=jax.lax.Precision.HIGHEST)
         + jnp.dot(agg, ca_ref[...], preferred_element_type=F32, precision=jax.lax.Precision.HIGHEST) + e1_ref[...])
    h = jnp.maximum(h, 0.0)
    o_ref[...] = x + jnp.dot(h, u2_ref[...], preferred_element_type=F32, precision=jax.lax.Precision.HIGHEST) + u2b_ref[...]


def _node(x, s_parts, c_parts, cx, ca, e1, u2, u2b, block=2000):
    n = x.shape[0]
    wspec = pl.BlockSpec((H, H), lambda i: (0, 0))
    bspec = pl.BlockSpec((1, H), lambda i: (0, 0))
    return pl.pallas_call(
        _node_body,
        grid=(n // block,),
        in_specs=[
            pl.BlockSpec((block, H), lambda i: (i, 0)),
            pl.BlockSpec((2, block, H), lambda i: (0, i, 0)),
            pl.BlockSpec((2, block, H), lambda i: (0, i, 0)),
            wspec, wspec, bspec, wspec, bspec,
        ],
        out_specs=pl.BlockSpec((block, H), lambda i: (i, 0)),
        out_shape=jax.ShapeDtypeStruct((n, H), F32),
    )(x, s_parts, c_parts, cx, ca, e1.reshape(1, -1), u2, u2b.reshape(1, -1))


_CH = 80


def _sc_gather(row, col, t1, t2):
    e = row.shape[0]
    per_w = e // NW
    nch = per_w // _CH
    d1, d2 = t1.shape[1], t2.shape[1]

    def body(row_hbm, col_hbm, t1_hbm, t2_hbm, g1_hbm, g2_hbm,
             rowv, colv, buf1, buf2, sem1, sem2):
        wid = lax.axis_index("s") * NC + lax.axis_index("c")
        base = wid * per_w

        def step(i, carry):
            off = base + i * _CH
            pltpu.sync_copy(row_hbm.at[pl.ds(off, _CH)], rowv)
            pltpu.sync_copy(col_hbm.at[pl.ds(off, _CH)], colv)
            cp1 = pltpu.async_copy(t1_hbm.at[rowv], buf1, sem1)
            cp2 = pltpu.async_copy(t2_hbm.at[colv], buf2, sem2)
            cp1.wait()
            cp2.wait()
            pltpu.sync_copy(buf1, g1_hbm.at[pl.ds(off, _CH)])
            pltpu.sync_copy(buf2, g2_hbm.at[pl.ds(off, _CH)])
            return carry

        lax.fori_loop(0, nch, step, 0)

    run = pl.kernel(
        body,
        out_type=[
            jax.ShapeDtypeStruct((e, d1), t1.dtype),
            jax.ShapeDtypeStruct((e, d2), t2.dtype),
        ],
        mesh=_mesh(),
        scratch_types=[
            pltpu.VMEM((_CH,), jnp.int32),
            pltpu.VMEM((_CH,), jnp.int32),
            pltpu.VMEM((_CH, d1), t1.dtype),
            pltpu.VMEM((_CH, d2), t2.dtype),
            pltpu.SemaphoreType.DMA,
            pltpu.SemaphoreType.DMA,
        ],
    )
    return run(row, col, t1, t2)


def _sc_scatter(col, m, zeros_n):
    e, d = m.shape
    n = zeros_n.shape[0]
    per_w = e // NW
    nch = per_w // _CH
    rows_per_tile = n // NS

    def body(col_hbm, m_hbm, z_hbm, out_hbm, colv, mbuf, shared):
        cid = lax.axis_index("c")
        sid = lax.axis_index("s")
        wid = sid * NC + cid
        base = wid * per_w
        tbase = sid * rows_per_tile
        pltpu.sync_copy(z_hbm.at[pl.ds(tbase, rows_per_tile)],
                        shared.at[pl.ds(tbase, rows_per_tile)])
        plsc.subcore_barrier()

        def step(i, carry):
            off = base + i * _CH
            pltpu.sync_copy(col_hbm.at[pl.ds(off, _CH)], colv)
            pltpu.sync_copy(m_hbm.at[pl.ds(off, _CH)], mbuf)
            pltpu.sync_copy(mbuf, shared.at[colv], add=True)
            return carry

        lax.fori_loop(0, nch, step, 0)
        plsc.subcore_barrier()
        pltpu.sync_copy(shared.at[pl.ds(tbase, rows_per_tile)],
                        out_hbm.at[cid, pl.ds(tbase, rows_per_tile)])

    run = pl.kernel(
        body,
        out_type=jax.ShapeDtypeStruct((NC, n, d), F32),
        mesh=_mesh(),
        scratch_types=[
            pltpu.VMEM((_CH,), jnp.int32),
            pltpu.VMEM((_CH, d), F32),
            pltpu.VMEM_SHARED((n, d), F32),
        ],
    )
    return run(col, m, zeros_n)


def _sc_count(col, ones_ch, zeros_n):
    e = col.shape[0]
    n, d = zeros_n.shape
    per_w = e // NW
    nch = per_w // _CH
    rows_per_tile = n // NS

    def body(col_hbm, ones_hbm, z_hbm, out_hbm, colv, onesv, shared):
        cid = lax.axis_index("c")
        sid = lax.axis_index("s")
        wid = sid * NC + cid
        base = wid * per_w
        tbase = sid * rows_per_tile
        pltpu.sync_copy(z_hbm.at[pl.ds(tbase, rows_per_tile)],
                        shared.at[pl.ds(tbase, rows_per_tile)])
        pltpu.sync_copy(ones_hbm, onesv)
        plsc.subcore_barrier()

        def step(i, carry):
            off = base + i * _CH
            pltpu.sync_copy(col_hbm.at[pl.ds(off, _CH)], colv)
            pltpu.sync_copy(onesv, shared.at[colv], add=True)
            return carry

        lax.fori_loop(0, nch, step, 0)
        plsc.subcore_barrier()
        pltpu.sync_copy(shared.at[pl.ds(tbase, rows_per_tile)],
                        out_hbm.at[cid, pl.ds(tbase, rows_per_tile)])

    run = pl.kernel(
        body,
        out_type=jax.ShapeDtypeStruct((NC, n, d), F32),
        mesh=_mesh(),
        scratch_types=[
            pltpu.VMEM((_CH,), jnp.int32),
            pltpu.VMEM((_CH, d), F32),
            pltpu.VMEM_SHARED((n, d), F32),
        ],
    )
    return run(col, ones_ch, zeros_n)


def kernel(x, edge_index, edge_attr, u, batch, params):
    p = params
    n = x.shape[0]
    e = edge_index.shape[1]
    row = edge_index[0]
    col = edge_index[1]

    ne = p["node_enc"]
    ee = p["edge_enc"]
    ge = p["glob_enc"]
    nd = p["node_dec"]

    x1 = _mlp2(x, ne[0]["W"], ne[0]["b"], ne[1]["W"], ne[1]["b"], block=2000)
    ea = _mlp2(edge_attr, ee[0]["W"], ee[0]["b"], ee[1]["W"], ee[1]["b"], block=3200)
    u_pad = jnp.pad(u, ((0, 7), (0, 0)))
    u0 = _mlp2(u_pad, ge[0]["W"], ge[0]["b"], ge[1]["W"], ge[1]["b"], block=8)[0:1]

    n_pad = ((n + NS * 8 - 1) // (NS * 8)) * (NS * 8)
    ones_ch = jnp.ones((_CH, H), F32)
    zeros_n = jnp.zeros((n_pad, H), F32)
    c_parts = _sc_count(col, ones_ch, zeros_n)

    for lp in p["layers"]:
        w1 = lp["edge_mlp"][0]["W"]
        b1 = lp["edge_mlp"][0]["b"]
        w2 = lp["edge_mlp"][1]["W"]
        b2 = lp["edge_mlp"][1]["b"]
        v1 = lp["node_mlp_1"][0]["W"]
        d1 = lp["node_mlp_1"][0]["b"]
        v2 = lp["node_mlp_1"][1]["W"]
        v2b = lp["node_mlp_1"][1]["b"]
        u1 = lp["node_mlp_2"][0]["W"]
        e1b = lp["node_mlp_2"][0]["b"]
        u2 = lp["node_mlp_2"][1]["W"]
        u2b = lp["node_mlp_2"][1]["b"]

        wa = jnp.concatenate([w1[:H], v1[:H]], axis=1)
        t1, t2 = _pre(x1, wa, w1[H:2 * H])
        t1i = jax.lax.bitcast_convert_type(t1.reshape(n, H, 2), jnp.int32)
        g1i, g2 = _sc_gather(row, col, t1i, t2)
        g1 = jax.lax.bitcast_convert_type(g1i, jnp.bfloat16).reshape(e, 2 * H)

        c1 = (u0 @ w1[3 * H:] + b1).reshape(-1)
        ea_next, m = _edge(g1, g2, ea, w1[2 * H:3 * H], c1, w2, b2,
                           v1[H:], d1, v2, v2b)
        s_parts = _sc_scatter(col, m, zeros_n)
        e1 = (u0 @ u1[2 * H:] + e1b).reshape(-1)
        x1 = _node(x1, s_parts, c_parts, u1[:H], u1[H:2 * H], e1, u2, u2b)
        ea = ea_next

    return _mlp2(x1, nd[0]["W"], nd[0]["b"], nd[1]["W"], nd[1]["b"], block=2000)


# speedup vs baseline: 1.9344x; 1.9344x over previous
import functools
import jax
import jax.numpy as jnp
from jax import lax
from jax.experimental import pallas as pl
from jax.experimental.pallas import tpu as pltpu
from jax.experimental.pallas import tpu_sc as plsc

H = 128
NC = 2
NS = 16
NW = NC * NS

F32 = jnp.float32


def _mesh():
    return plsc.VectorSubcoreMesh(
        core_axis_name="c", subcore_axis_name="s", num_cores=NC, num_subcores=NS
    )


def _mlp2_body(x_ref, w1_ref, b1_ref, w2_ref, b2_ref, o_ref):
    h = jnp.dot(x_ref[...], w1_ref[...], preferred_element_type=F32, precision=jax.lax.Precision.HIGHEST) + b1_ref[...]
    h = jnp.maximum(h, 0.0)
    o_ref[...] = jnp.dot(h, w2_ref[...], preferred_element_type=F32, precision=jax.lax.Precision.HIGHEST) + b2_ref[...]


def _mlp2(x, w1, b1, w2, b2, block):
    r, k = x.shape
    hh = w1.shape[1]
    f = w2.shape[1]
    return pl.pallas_call(
        _mlp2_body,
        grid=(r // block,),
        in_specs=[
            pl.BlockSpec((block, k), lambda i: (i, 0)),
            pl.BlockSpec((k, hh), lambda i: (0, 0)),
            pl.BlockSpec((1, hh), lambda i: (0, 0)),
            pl.BlockSpec((hh, f), lambda i: (0, 0)),
            pl.BlockSpec((1, f), lambda i: (0, 0)),
        ],
        out_specs=pl.BlockSpec((block, f), lambda i: (i, 0)),
        out_shape=jax.ShapeDtypeStruct((r, f), F32),
    )(x, w1, b1.reshape(1, -1), w2, b2.reshape(1, -1))


def _pre_body(x_ref, wa_ref, wb_ref, t1_ref, t2_ref):
    x = x_ref[...]
    t1_ref[...] = jnp.dot(x, wa_ref[...], preferred_element_type=F32, precision=jax.lax.Precision.HIGHEST)
    t2_ref[...] = jnp.dot(x, wb_ref[...], preferred_element_type=F32, precision=jax.lax.Precision.HIGHEST)


def _pre(x, wa, wb, block=2000):
    n = x.shape[0]
    return pl.pallas_call(
        _pre_body,
        grid=(n // block,),
        in_specs=[
            pl.BlockSpec((block, H), lambda i: (i, 0)),
            pl.BlockSpec((H, 2 * H), lambda i: (0, 0)),
            pl.BlockSpec((H, H), lambda i: (0, 0)),
        ],
        out_specs=[
            pl.BlockSpec((block, 2 * H), lambda i: (i, 0)),
            pl.BlockSpec((block, H), lambda i: (i, 0)),
        ],
        out_shape=[
            jax.ShapeDtypeStruct((n, 2 * H), F32),
            jax.ShapeDtypeStruct((n, H), F32),
        ],
    )(x, wa, wb)


def _edge_body(g1_ref, g2_ref, ea_ref, we_ref, c1_ref, w2_ref, b2_ref,
               ve_ref, d1_ref, v2_ref, v2b_ref, eanext_ref, m_ref):
    ea = ea_ref[...]
    g1a = g1_ref[:, :H]
    g1b = g1_ref[:, H:]
    g2 = g2_ref[...]
    h = g1a + g2 + jnp.dot(ea, we_ref[...], preferred_element_type=F32, precision=jax.lax.Precision.HIGHEST) + c1_ref[...]
    h = jnp.maximum(h, 0.0)
    ea_new = jnp.dot(h, w2_ref[...], preferred_element_type=F32, precision=jax.lax.Precision.HIGHEST) + b2_ref[...]
    t = g1b + jnp.dot(ea_new, ve_ref[...], preferred_element_type=F32, precision=jax.lax.Precision.HIGHEST) + d1_ref[...]
    t = jnp.maximum(t, 0.0)
    m_ref[...] = jnp.dot(t, v2_ref[...], preferred_element_type=F32, precision=jax.lax.Precision.HIGHEST) + v2b_ref[...]
    eanext_ref[...] = ea_new + ea


def _edge(g1, g2, ea, we, c1, w2, b2, ve, d1, v2, v2b, block=3200):
    e = ea.shape[0]
    wspec = pl.BlockSpec((H, H), lambda i: (0, 0))
    bspec = pl.BlockSpec((1, H), lambda i: (0, 0))
    return pl.pallas_call(
        _edge_body,
        grid=(e // block,),
        in_specs=[
            pl.BlockSpec((block, 2 * H), lambda i: (i, 0)),
            pl.BlockSpec((block, H), lambda i: (i, 0)),
            pl.BlockSpec((block, H), lambda i: (i, 0)),
            wspec, bspec, wspec, bspec, wspec, bspec, wspec, bspec,
        ],
        out_specs=[
            pl.BlockSpec((block, H), lambda i: (i, 0)),
            pl.BlockSpec((block, H), lambda i: (i, 0)),
        ],
        out_shape=[
            jax.ShapeDtypeStruct((e, H), F32),
            jax.ShapeDtypeStruct((e, H), F32),
        ],
    )(g1, g2, ea, we, c1.reshape(1, -1), w2, b2.reshape(1, -1),
      ve, d1.reshape(1, -1), v2, v2b.reshape(1, -1))


def _node_body(x_ref, s_ref, c_ref, cx_ref, ca_ref, e1_ref, u2_ref, u2b_ref, o_ref):
    s = s_ref[0] + s_ref[1]
    cnt = c_ref[0, :, 0:1] + c_ref[1, :, 0:1]
    agg = s / jnp.maximum(cnt, 1.0)
    x = x_ref[...]
    h = (jnp.dot(x, cx_ref[...], preferred_element_type=F32, precision=jax.lax.Precision.HIGHEST)
         + jnp.dot(agg, ca_ref[...], preferred_element_type=F32, precision=jax.lax.Precision.HIGHEST) + e1_ref[...])
    h = jnp.maximum(h, 0.0)
    o_ref[...] = x + jnp.dot(h, u2_ref[...], preferred_element_type=F32, precision=jax.lax.Precision.HIGHEST) + u2b_ref[...]


def _node(x, s_parts, c_parts, cx, ca, e1, u2, u2b, block=2000):
    n = x.shape[0]
    wspec = pl.BlockSpec((H, H), lambda i: (0, 0))
    bspec = pl.BlockSpec((1, H), lambda i: (0, 0))
    return pl.pallas_call(
        _node_body,
        grid=(n // block,),
        in_specs=[
            pl.BlockSpec((block, H), lambda i: (i, 0)),
            pl.BlockSpec((2, block, H), lambda i: (0, i, 0)),
            pl.BlockSpec((2, block, H), lambda i: (0, i, 0)),
            wspec, wspec, bspec, wspec, bspec,
        ],
        out_specs=pl.BlockSpec((block, H), lambda i: (i, 0)),
        out_shape=jax.ShapeDtypeStruct((n, H), F32),
    )(x, s_parts, c_parts, cx, ca, e1.reshape(1, -1), u2, u2b.reshape(1, -1))


_CH = 40
_NBUF = 5


def _sc_gather(row, col, t1, t2):
    e = row.shape[0]
    per_w = e // NW
    nch = per_w // _CH
    ngrp = nch // _NBUF
    t1_row = t1.shape[1:]
    t2_row = t2.shape[1:]

    def body(row_hbm, col_hbm, t1_hbm, t2_hbm, g1_hbm, g2_hbm,
             rowv, colv, *scr):
        bufs1 = scr[0:_NBUF]
        bufs2 = scr[_NBUF:2 * _NBUF]
        semg = scr[2 * _NBUF:3 * _NBUF]
        semw = scr[3 * _NBUF:4 * _NBUF]
        wid = lax.axis_index("s") * NC + lax.axis_index("c")
        base = wid * per_w
        pltpu.sync_copy(row_hbm.at[pl.ds(base, per_w)], rowv)
        pltpu.sync_copy(col_hbm.at[pl.ds(base, per_w)], colv)

        def g_start(p, j):
            off = j * _CH
            pltpu.async_copy(t1_hbm.at[rowv.at[pl.ds(off, _CH)]], bufs1[p], semg[p])
            pltpu.async_copy(t2_hbm.at[colv.at[pl.ds(off, _CH)]], bufs2[p], semg[p])

        def g_wait(p):
            pltpu.make_async_copy(t1_hbm.at[rowv.at[pl.ds(0, _CH)]], bufs1[p], semg[p]).wait()
            pltpu.make_async_copy(t2_hbm.at[colv.at[pl.ds(0, _CH)]], bufs2[p], semg[p]).wait()

        def w_start(p, j):
            off = base + j * _CH
            pltpu.async_copy(bufs1[p], g1_hbm.at[pl.ds(off, _CH)], semw[p])
            pltpu.async_copy(bufs2[p], g2_hbm.at[pl.ds(off, _CH)], semw[p])

        def w_wait(p):
            pltpu.make_async_copy(bufs1[p], g1_hbm.at[pl.ds(base, _CH)], semw[p]).wait()
            pltpu.make_async_copy(bufs2[p], g2_hbm.at[pl.ds(base, _CH)], semw[p]).wait()

        for p in range(_NBUF):
            g_start(p, p)

        def grp(i, carry):
            jb = i * _NBUF
            for p in range(_NBUF):
                g_wait(p)
            for p in range(_NBUF):
                w_start(p, jb + p)
            for p in range(_NBUF):
                w_wait(p)

            @pl.when(i < ngrp - 1)
            def _():
                for p in range(_NBUF):
                    g_start(p, jb + _NBUF + p)

            return carry

        lax.fori_loop(0, ngrp, grp, 0)

    run = pl.kernel(
        body,
        out_type=[
            jax.ShapeDtypeStruct((e,) + t1_row, t1.dtype),
            jax.ShapeDtypeStruct((e,) + t2_row, t2.dtype),
        ],
        mesh=_mesh(),
        scratch_types=(
            [pltpu.VMEM((per_w,), jnp.int32), pltpu.VMEM((per_w,), jnp.int32)]
            + [pltpu.VMEM((_CH,) + t1_row, t1.dtype) for _ in range(_NBUF)]
            + [pltpu.VMEM((_CH,) + t2_row, t2.dtype) for _ in range(_NBUF)]
            + [pltpu.SemaphoreType.DMA for _ in range(2 * _NBUF)]
        ),
    )
    return run(row, col, t1, t2)


def _sc_scatter(col, m, zeros_n):
    e, d = m.shape
    n = zeros_n.shape[0]
    per_w = e // NW
    nch = per_w // _CH
    rows_per_tile = n // NS

    nbuf = 2
    ngrp = nch // nbuf
    col3 = col.reshape(NW, nch, _CH)

    def body(col_hbm, m_hbm, z_hbm, out_hbm, colv, *scr):
        mbufs = scr[0:nbuf]
        semm = scr[nbuf:2 * nbuf]
        shared = scr[2 * nbuf]
        cid = lax.axis_index("c")
        sid = lax.axis_index("s")
        wid = sid * NC + cid
        base = wid * per_w
        tbase = sid * rows_per_tile
        pltpu.sync_copy(z_hbm.at[pl.ds(tbase, rows_per_tile)],
                        shared.at[pl.ds(tbase, rows_per_tile)])
        pltpu.sync_copy(col_hbm.at[wid], colv)
        plsc.subcore_barrier()

        def m_start(p, j):
            pltpu.async_copy(m_hbm.at[pl.ds(base + j * _CH, _CH)], mbufs[p], semm[p])

        def m_wait(p):
            pltpu.make_async_copy(m_hbm.at[pl.ds(base, _CH)], mbufs[p], semm[p]).wait()

        for p in range(nbuf):
            m_start(p, p)

        def grp(i, carry):
            jb = i * nbuf
            for p in range(nbuf):
                m_wait(p)
                pltpu.sync_copy(mbufs[p], shared.at[colv.at[jb + p]], add=True)

                @pl.when(i < ngrp - 1)
                def _():
                    m_start(p, jb + nbuf + p)

            return carry

        lax.fori_loop(0, ngrp, grp, 0)
        plsc.subcore_barrier()
        pltpu.sync_copy(shared.at[pl.ds(tbase, rows_per_tile)],
                        out_hbm.at[cid, pl.ds(tbase, rows_per_tile)])

    run = pl.kernel(
        body,
        out_type=jax.ShapeDtypeStruct((NC, n, d), F32),
        mesh=_mesh(),
        scratch_types=(
            [pltpu.VMEM((nch, _CH), jnp.int32)]
            + [pltpu.VMEM((_CH, d), F32) for _ in range(nbuf)]
            + [pltpu.SemaphoreType.DMA for _ in range(nbuf)]
            + [pltpu.VMEM_SHARED((n, d), F32)]
        ),
    )
    return run(col3, m, zeros_n)


def _sc_count(col, ones_ch, zeros_n):
    e = col.shape[0]
    n, d = zeros_n.shape
    per_w = e // NW
    nch = per_w // _CH
    rows_per_tile = n // NS

    def body(col_hbm, ones_hbm, z_hbm, out_hbm, colv, onesv, shared):
        cid = lax.axis_index("c")
        sid = lax.axis_index("s")
        wid = sid * NC + cid
        base = wid * per_w
        tbase = sid * rows_per_tile
        pltpu.sync_copy(z_hbm.at[pl.ds(tbase, rows_per_tile)],
                        shared.at[pl.ds(tbase, rows_per_tile)])
        pltpu.sync_copy(ones_hbm, onesv)
        plsc.subcore_barrier()

        def step(i, carry):
            off = base + i * _CH
            pltpu.sync_copy(col_hbm.at[pl.ds(off, _CH)], colv)
            pltpu.sync_copy(onesv, shared.at[colv], add=True)
            return carry

        lax.fori_loop(0, nch, step, 0)
        plsc.subcore_barrier()
        pltpu.sync_copy(shared.at[pl.ds(tbase, rows_per_tile)],
                        out_hbm.at[cid, pl.ds(tbase, rows_per_tile)])

    run = pl.kernel(
        body,
        out_type=jax.ShapeDtypeStruct((NC, n, d), F32),
        mesh=_mesh(),
        scratch_types=[
            pltpu.VMEM((_CH,), jnp.int32),
            pltpu.VMEM((_CH, d), F32),
            pltpu.VMEM_SHARED((n, d), F32),
        ],
    )
    return run(col, ones_ch, zeros_n)


def kernel(x, edge_index, edge_attr, u, batch, params):
    p = params
    n = x.shape[0]
    e = edge_index.shape[1]
    row = edge_index[0]
    col = edge_index[1]

    ne = p["node_enc"]
    ee = p["edge_enc"]
    ge = p["glob_enc"]
    nd = p["node_dec"]

    x1 = _mlp2(x, ne[0]["W"], ne[0]["b"], ne[1]["W"], ne[1]["b"], block=2000)
    ea = _mlp2(edge_attr, ee[0]["W"], ee[0]["b"], ee[1]["W"], ee[1]["b"], block=3200)
    u_pad = jnp.pad(u, ((0, 7), (0, 0)))
    u0 = _mlp2(u_pad, ge[0]["W"], ge[0]["b"], ge[1]["W"], ge[1]["b"], block=8)[0:1]

    n_pad = ((n + NS * 8 - 1) // (NS * 8)) * (NS * 8)
    ones_ch = jnp.ones((_CH, H), F32)
    zeros_n = jnp.zeros((n_pad, H), F32)
    c_parts = _sc_count(col, ones_ch, zeros_n)

    for lp in p["layers"]:
        w1 = lp["edge_mlp"][0]["W"]
        b1 = lp["edge_mlp"][0]["b"]
        w2 = lp["edge_mlp"][1]["W"]
        b2 = lp["edge_mlp"][1]["b"]
        v1 = lp["node_mlp_1"][0]["W"]
        d1 = lp["node_mlp_1"][0]["b"]
        v2 = lp["node_mlp_1"][1]["W"]
        v2b = lp["node_mlp_1"][1]["b"]
        u1 = lp["node_mlp_2"][0]["W"]
        e1b = lp["node_mlp_2"][0]["b"]
        u2 = lp["node_mlp_2"][1]["W"]
        u2b = lp["node_mlp_2"][1]["b"]

        wa = jnp.concatenate([w1[:H], v1[:H]], axis=1)
        t1, t2 = _pre(x1, wa, w1[H:2 * H])
        g1, g2 = _sc_gather(row, col, t1, t2)

        c1 = (u0 @ w1[3 * H:] + b1).reshape(-1)
        ea_next, m = _edge(g1, g2, ea, w1[2 * H:3 * H], c1, w2, b2,
                           v1[H:], d1, v2, v2b)
        s_parts = _sc_scatter(col, m, zeros_n)
        e1 = (u0 @ u1[2 * H:] + e1b).reshape(-1)
        x1 = _node(x1, s_parts, c_parts, u1[:H], u1[H:2 * H], e1, u2, u2b)
        ea = ea_next

    return _mlp2(x1, nd[0]["W"], nd[0]["b"], nd[1]["W"], nd[1]["b"], block=2000)
